# baseline (speedup 1.0000x reference)
import functools

import jax
import jax.numpy as jnp
from jax import lax
from jax.experimental import pallas as pl
from jax.experimental.pallas import tpu as pltpu
from jax.experimental.pallas import tpu_sc as plsc

_FIELD = 26
_D = 16
_BATCH = 16384
_VOCAB = 40000
_TOTAL = _FIELD * _VOCAB
_NC = 2
_NS = 16
_NW = _NC * _NS
_RPW = _BATCH // _NW
_CHUNK = 64
_NCHUNK = _RPW // _CHUNK
_IPC = _CHUNK * _FIELD


def _fm_body(emb, fcf, idx, out, idx_v, ixp_v, rows_v, fc_v, out_v, gsem):
    wid = lax.axis_index("s") * _NC + lax.axis_index("c")
    row_base = wid * _RPW
    lane = lax.iota(jnp.int32, 16)
    dstride = lane * _TOTAL

    pltpu.sync_copy(idx.at[:, pl.ds(row_base, _RPW)], idx_v)

    def chunk_body(c, carry):
        def expand_body(r, carry2):
            for f in range(_FIELD):
                v = plsc.load_gather(
                    idx_v, [jnp.full((16,), f, jnp.int32),
                            jnp.full((16,), c * _CHUNK, jnp.int32) + r])
                ixp_v[pl.ds((f * _CHUNK + r) * _D, _D)] = v + dstride
            return carry2

        lax.fori_loop(0, _CHUNK, expand_body, 0)

        copies = []
        for f in range(_FIELD):
            iv = idx_v.at[f, pl.ds(c * _CHUNK, _CHUNK)]
            copies.append(
                pltpu.async_copy(fcf.at[iv],
                                 fc_v.at[pl.ds(f * _CHUNK, _CHUNK)], gsem))
            for sub in range(_CHUNK * _D // 128):
                base = f * _CHUNK * _D + sub * 128
                copies.append(
                    pltpu.async_copy(emb.at[ixp_v.at[pl.ds(base, 128)]],
                                     rows_v.at[pl.ds(base, 128)], gsem))
        for h in copies:
            h.wait()

        def grp_body(g, carry2):
            row0 = lane + g * 16
            rows = [row0 + f * _CHUNK for f in range(_FIELD)]
            rows16 = [r * _D for r in rows]
            facc = plsc.load_gather(fc_v, [rows[0]])
            for f in range(1, _FIELD):
                facc = facc + plsc.load_gather(fc_v, [rows[f]])
            acc = jnp.zeros((16,), jnp.float32)
            for d in range(_D):
                s = jnp.zeros((16,), jnp.float32)
                ss = jnp.zeros((16,), jnp.float32)
                for f in range(_FIELD):
                    v = plsc.load_gather(rows_v, [rows16[f] + d])
                    s = s + v
                    ss = ss + v * v
                acc = acc + (s * s - ss)
            out_v[pl.ds(g * 16, 16)] = facc + 0.5 * acc
            return carry2

        lax.fori_loop(0, _CHUNK // 16, grp_body, 0)
        pltpu.sync_copy(out_v, out.at[pl.ds(row_base + c * _CHUNK, _CHUNK)])
        return carry

    lax.fori_loop(0, _NCHUNK, chunk_body, 0)


_fm = functools.partial(
    pl.kernel,
    out_type=jax.ShapeDtypeStruct((_BATCH,), jnp.float32),
    mesh=plsc.VectorSubcoreMesh(
        core_axis_name="c", subcore_axis_name="s",
        num_cores=_NC, num_subcores=_NS),
    compiler_params=pltpu.CompilerParams(
        needs_layout_passes=False, use_tc_tiling_on_sc=False),
    scratch_types=[
        pltpu.VMEM((_FIELD, _RPW), jnp.int32),
        pltpu.VMEM((_IPC * _D,), jnp.int32),
        pltpu.VMEM((_IPC * _D,), jnp.float32),
        pltpu.VMEM((_IPC,), jnp.float32),
        pltpu.VMEM((_CHUNK,), jnp.float32),
        pltpu.SemaphoreType.DMA,
    ],
)(_fm_body)


@jax.jit
def kernel(x, emb_table, fc_table, bias):
    offs = jnp.arange(_FIELD, dtype=jnp.int32) * _VOCAB
    idx_t = x.T + offs[:, None]
    emb_flat = emb_table.T.reshape(-1)
    out = _fm(emb_flat, fc_table.reshape(-1), idx_t)
    return out + bias[0]


# speedup vs baseline: 6.2771x; 6.2771x over previous
import functools

import jax
import jax.numpy as jnp
from jax import lax
from jax.experimental import pallas as pl
from jax.experimental.pallas import tpu as pltpu
from jax.experimental.pallas import tpu_sc as plsc

_FIELD = 26
_D = 16
_BATCH = 16384
_VOCAB = 40000
_TOTAL = _FIELD * _VOCAB
_NC = 2
_NS = 16
_NW = _NC * _NS
_RPW = _BATCH // _NW
_CHUNK = 64
_NCHUNK = _RPW // _CHUNK
_IPC = _CHUNK * _FIELD

_TBLK = 1024
_NBLK = _TOTAL // _TBLK
_TAIL = _TOTAL - _NBLK * _TBLK
_SLOTS = (_NBLK + _NW - 1) // _NW


def _tr_transpose_block(pad_v, rowr, nrows):
    lane16 = lax.iota(jnp.int32, 16) * _D

    def g_body(g, carry):
        dst = lane16 + g * (16 * _D)
        for i in range(2):
            for r in range(8):
                v = pad_v[i, r, pl.ds(g * 16, 16)]
                plsc.store_scatter(rowr, [dst + (8 * i + r)], v)
        return carry

    lax.fori_loop(0, nrows // 16, g_body, 0)


def _tr_body(embt, outt, pad_a, pad_b, row_v, sem):
    wid = lax.axis_index("s") * _NC + lax.axis_index("c")
    bufs = (pad_a, pad_b)

    def descs(blk, buf, fire):
        fn = pltpu.async_copy if fire else pltpu.make_async_copy
        return [
            fn(embt.at[pl.ds(8 * i, 8), pl.ds(blk * _TBLK, _TBLK)],
               buf.at[i], sem)
            for i in range(2)
        ]

    for s in range(_SLOTS + 1):
        if s < _SLOTS:
            blk = s * _NW + wid

            @pl.when(blk < _NBLK)
            def _fire(blk=blk, buf=bufs[s % 2]):
                descs(blk, buf, True)

        if s > 0:
            blk_p = (s - 1) * _NW + wid

            @pl.when(blk_p < _NBLK)
            def _proc(blk_p=blk_p, buf=bufs[(s - 1) % 2]):
                for h in descs(blk_p, buf, False):
                    h.wait()
                _tr_transpose_block(buf, row_v, _TBLK)
                pltpu.sync_copy(
                    row_v, outt.at[pl.ds(blk_p * _TBLK * _D, _TBLK * _D)])

    @pl.when(wid == _NW - 1)
    def _tail():
        base = _NBLK * _TBLK
        copies = [
            pltpu.async_copy(embt.at[pl.ds(8 * i, 8), pl.ds(base, _TAIL)],
                             pad_a.at[i, :, pl.ds(0, _TAIL)], sem)
            for i in range(2)
        ]
        for h in copies:
            h.wait()
        _tr_transpose_block(pad_a, row_v, _TAIL)
        pltpu.sync_copy(row_v.at[pl.ds(0, _TAIL * _D)],
                        outt.at[pl.ds(base * _D, _TAIL * _D)])


def _fm_body(emb, fcf, idx, out, idx_v, rows_v, fc_v, out_v, gsem):
    wid = lax.axis_index("s") * _NC + lax.axis_index("c")
    row_base = wid * _RPW
    lane = lax.iota(jnp.int32, 16)

    pltpu.sync_copy(idx.at[:, pl.ds(row_base, _RPW)], idx_v)

    def chunk_body(c, carry):
        copies = []
        for f in range(_FIELD):
            iv = idx_v.at[f, pl.ds(c * _CHUNK, _CHUNK)]
            copies.append(
                pltpu.async_copy(emb.at[iv],
                                 rows_v.at[pl.ds(f * _CHUNK, _CHUNK)], gsem))
            copies.append(
                pltpu.async_copy(fcf.at[iv],
                                 fc_v.at[pl.ds(f * _CHUNK, _CHUNK)], gsem))
        for h in copies:
            h.wait()

        def grp_body(g, carry2):
            row0 = lane + g * 16
            rows = [row0 + f * _CHUNK for f in range(_FIELD)]
            facc = plsc.load_gather(fc_v, [rows[0]])
            for f in range(1, _FIELD):
                facc = facc + plsc.load_gather(fc_v, [rows[f]])
            acc = jnp.zeros((16,), jnp.float32)
            for d in range(_D):
                col = jnp.full((16,), d, jnp.int32)
                s = jnp.zeros((16,), jnp.float32)
                ss = jnp.zeros((16,), jnp.float32)
                for f in range(_FIELD):
                    v = plsc.load_gather(rows_v, [rows[f], col])
                    s = s + v
                    ss = ss + v * v
                acc = acc + (s * s - ss)
            out_v[pl.ds(g * 16, 16)] = facc + 0.5 * acc
            return carry2

        lax.fori_loop(0, _CHUNK // 16, grp_body, 0)
        pltpu.sync_copy(out_v, out.at[pl.ds(row_base + c * _CHUNK, _CHUNK)])
        return carry

    lax.fori_loop(0, _NCHUNK, chunk_body, 0)


_MESH = plsc.VectorSubcoreMesh(
    core_axis_name="c", subcore_axis_name="s",
    num_cores=_NC, num_subcores=_NS)
_PARAMS = pltpu.CompilerParams(
    needs_layout_passes=False, use_tc_tiling_on_sc=False)

_transpose = functools.partial(
    pl.kernel,
    out_type=jax.ShapeDtypeStruct((_TOTAL * _D,), jnp.float32),
    mesh=_MESH,
    compiler_params=pltpu.CompilerParams(
        needs_layout_passes=False, use_tc_tiling_on_sc=True),
    scratch_types=[
        pltpu.VMEM((2, 8, _TBLK), jnp.float32),
        pltpu.VMEM((2, 8, _TBLK), jnp.float32),
        pltpu.VMEM((_TBLK * _D,), jnp.float32),
        pltpu.SemaphoreType.DMA,
    ],
)(_tr_body)

_fm = functools.partial(
    pl.kernel,
    out_type=jax.ShapeDtypeStruct((_BATCH,), jnp.float32),
    mesh=_MESH,
    compiler_params=_PARAMS,
    scratch_types=[
        pltpu.VMEM((_FIELD, _RPW), jnp.int32),
        pltpu.VMEM((_IPC, _D), jnp.float32),
        pltpu.VMEM((_IPC,), jnp.float32),
        pltpu.VMEM((_CHUNK,), jnp.float32),
        pltpu.SemaphoreType.DMA,
    ],
)(_fm_body)


@jax.jit
def kernel(x, emb_table, fc_table, bias):
    offs = jnp.arange(_FIELD, dtype=jnp.int32) * _VOCAB
    idx_t = x.T + offs[:, None]
    emb_rows = _transpose(emb_table.T).reshape(_TOTAL, _D)
    out = _fm(emb_rows, fc_table.reshape(-1), idx_t)
    return out + bias[0]


# speedup vs baseline: 7.3097x; 1.1645x over previous
import functools

import jax
import jax.numpy as jnp
from jax import lax
from jax.experimental import pallas as pl
from jax.experimental.pallas import tpu as pltpu
from jax.experimental.pallas import tpu_sc as plsc

_FIELD = 26
_D = 16
_BATCH = 16384
_VOCAB = 40000
_TOTAL = _FIELD * _VOCAB
_NC = 2
_NS = 16
_NW = _NC * _NS
_RPW = _BATCH // _NW
_CHUNK = 64
_NCHUNK = _RPW // _CHUNK
_IPC = _CHUNK * _FIELD

_TBLK = 1024
_NBLK = _TOTAL // _TBLK
_TAIL = _TOTAL - _NBLK * _TBLK
_SLOTS = (_NBLK + _NW - 1) // _NW


def _tr_transpose_block(pad_v, rowr, nrows):
    lane16 = lax.iota(jnp.int32, 16) * _D

    def g_body(g, carry):
        dst = lane16 + g * (16 * _D)
        for i in range(2):
            for r in range(8):
                v = pad_v[i, r, pl.ds(g * 16, 16)]
                plsc.store_scatter(rowr, [dst + (8 * i + r)], v)
        return carry

    lax.fori_loop(0, nrows // 16, g_body, 0)


def _tr_body(embt, outt, pad_a, pad_b, row_a, row_b, sem, osem):
    wid = lax.axis_index("s") * _NC + lax.axis_index("c")
    bufs = (pad_a, pad_b)
    rbufs = (row_a, row_b)

    def descs(blk, buf, fire):
        fn = pltpu.async_copy if fire else pltpu.make_async_copy
        return [
            fn(embt.at[pl.ds(8 * i, 8), pl.ds(blk * _TBLK, _TBLK)],
               buf.at[i], sem)
            for i in range(2)
        ]

    def odesc(blk, rbuf, fire):
        fn = pltpu.async_copy if fire else pltpu.make_async_copy
        return fn(rbuf, outt.at[pl.ds(blk * _TBLK * _D, _TBLK * _D)], osem)

    for s in range(_SLOTS + 1):
        if s < _SLOTS:
            blk = s * _NW + wid

            @pl.when(blk < _NBLK)
            def _fire(blk=blk, buf=bufs[s % 2]):
                descs(blk, buf, True)

        if s > 0:
            blk_p = (s - 1) * _NW + wid

            @pl.when(blk_p < _NBLK)
            def _proc(blk_p=blk_p, buf=bufs[(s - 1) % 2],
                      rbuf=rbufs[(s - 1) % 2], s=s):
                for h in descs(blk_p, buf, False):
                    h.wait()
                if s >= 3:
                    odesc(blk_p - 2 * _NW, rbuf, False).wait()
                _tr_transpose_block(buf, rbuf, _TBLK)
                odesc(blk_p, rbuf, True)

    for s2 in (_SLOTS - 3, _SLOTS - 2, _SLOTS - 1):
        blk2 = s2 * _NW + wid

        @pl.when((blk2 < _NBLK) & (blk2 + 2 * _NW >= _NBLK))
        def _drain(blk2=blk2, rbuf=rbufs[s2 % 2]):
            odesc(blk2, rbuf, False).wait()

    @pl.when(wid == _NW - 1)
    def _tail():
        base = _NBLK * _TBLK
        copies = [
            pltpu.async_copy(embt.at[pl.ds(8 * i, 8), pl.ds(base, _TAIL)],
                             pad_a.at[i, :, pl.ds(0, _TAIL)], sem)
            for i in range(2)
        ]
        for h in copies:
            h.wait()
        _tr_transpose_block(pad_a, row_a, _TAIL)
        pltpu.sync_copy(row_a.at[pl.ds(0, _TAIL * _D)],
                        outt.at[pl.ds(base * _D, _TAIL * _D)])


_CBYTES = _FIELD * (_CHUNK * _D * 4 + _CHUNK * 4)


def _fm_body(emb, fcf, idx, out, idx_v, rows_a, rows_b, fc_a, fc_b, out_v,
             sem_a, sem_b):
    wid = lax.axis_index("s") * _NC + lax.axis_index("c")
    row_base = wid * _RPW
    lane = lax.iota(jnp.int32, 16)
    bufs = ((rows_a, fc_a, sem_a), (rows_b, fc_b, sem_b))

    pltpu.sync_copy(idx.at[:, pl.ds(row_base, _RPW)], idx_v)

    def fire(c, rows_v, fc_v, sem, launch=True):
        fn = pltpu.async_copy if launch else pltpu.make_async_copy
        descs = []
        for f in range(_FIELD):
            iv = idx_v.at[f, pl.ds(c * _CHUNK, _CHUNK)]
            descs.append(fn(emb.at[iv],
                            rows_v.at[pl.ds(f * _CHUNK, _CHUNK)], sem))
            descs.append(fn(fcf.at[iv],
                            fc_v.at[pl.ds(f * _CHUNK, _CHUNK)], sem))
        return descs

    fire(0, *bufs[0])
    fire(1, *bufs[1])

    def loop_body(i, carry):
        for k in range(2):
            rows_v, fc_v, sem = bufs[k]
            c = 2 * i + k
            for h in fire(c, rows_v, fc_v, sem, launch=False):
                h.wait()

            def grp_body(g, carry2):
                row0 = lane + g * 16
                rows = [row0 + f * _CHUNK for f in range(_FIELD)]
                facc = plsc.load_gather(fc_v, [rows[0]])
                for f in range(1, _FIELD):
                    facc = facc + plsc.load_gather(fc_v, [rows[f]])
                acc = jnp.zeros((16,), jnp.float32)
                for d in range(_D):
                    col = jnp.full((16,), d, jnp.int32)
                    s = jnp.zeros((16,), jnp.float32)
                    ss = jnp.zeros((16,), jnp.float32)
                    for f in range(_FIELD):
                        v = plsc.load_gather(rows_v, [rows[f], col])
                        s = s + v
                        ss = ss + v * v
                    acc = acc + (s * s - ss)
                out_v[pl.ds(g * 16, 16)] = facc + 0.5 * acc
                return carry2

            lax.fori_loop(0, _CHUNK // 16, grp_body, 0)
            pltpu.sync_copy(out_v,
                            out.at[pl.ds(row_base + c * _CHUNK, _CHUNK)])

            @pl.when(c + 2 < _NCHUNK)
            def _refire(c=c, rows_v=rows_v, fc_v=fc_v, sem=sem):
                fire(c + 2, rows_v, fc_v, sem)
        return carry

    lax.fori_loop(0, _NCHUNK // 2, loop_body, 0)


_MESH = plsc.VectorSubcoreMesh(
    core_axis_name="c", subcore_axis_name="s",
    num_cores=_NC, num_subcores=_NS)
_PARAMS = pltpu.CompilerParams(
    needs_layout_passes=False, use_tc_tiling_on_sc=False)

_transpose = functools.partial(
    pl.kernel,
    out_type=jax.ShapeDtypeStruct((_TOTAL * _D,), jnp.float32),
    mesh=_MESH,
    compiler_params=pltpu.CompilerParams(
        needs_layout_passes=False, use_tc_tiling_on_sc=True),
    scratch_types=[
        pltpu.VMEM((2, 8, _TBLK), jnp.float32),
        pltpu.VMEM((2, 8, _TBLK), jnp.float32),
        pltpu.VMEM((_TBLK * _D,), jnp.float32),
        pltpu.VMEM((_TBLK * _D,), jnp.float32),
        pltpu.SemaphoreType.DMA,
        pltpu.SemaphoreType.DMA,
    ],
)(_tr_body)

_fm = functools.partial(
    pl.kernel,
    out_type=jax.ShapeDtypeStruct((_BATCH,), jnp.float32),
    mesh=_MESH,
    compiler_params=_PARAMS,
    scratch_types=[
        pltpu.VMEM((_FIELD, _RPW), jnp.int32),
        pltpu.VMEM((_IPC, _D), jnp.float32),
        pltpu.VMEM((_IPC, _D), jnp.float32),
        pltpu.VMEM((_IPC,), jnp.float32),
        pltpu.VMEM((_IPC,), jnp.float32),
        pltpu.VMEM((_CHUNK,), jnp.float32),
        pltpu.SemaphoreType.DMA,
        pltpu.SemaphoreType.DMA,
    ],
)(_fm_body)


@jax.jit
def kernel(x, emb_table, fc_table, bias):
    offs = jnp.arange(_FIELD, dtype=jnp.int32) * _VOCAB
    idx_t = x.T + offs[:, None]
    emb_rows = _transpose(emb_table.T).reshape(_TOTAL, _D)
    out = _fm(emb_rows, fc_table.reshape(-1), idx_t)
    return out + bias[0]
